# baseline (speedup 1.0000x reference)
import functools

import jax
import jax.numpy as jnp
from jax import lax
from jax.experimental import pallas as pl
from jax.experimental.pallas import tpu as pltpu
from jax.experimental.pallas import tpu_sc as plsc

N = 50000
IN_DIM = 128
H = 64
E = 800000

NC = 2
NS = 16
CH = 128
EPT = E // NS
NCH2 = 392
HALFCH = NCH2 // 2
HEDGES = HALFCH * CH
SLAB = 3584
NSLAB = HEDGES // SLAB
EPAD = E + 2 * CH
SCH = 96
SNCH = 524
SEPT = SNCH * SCH
SEPAD = NS * SEPT
HALF = N // NC
ACCR = HALF + 24
DUMMY = HALF
ZROWS = 46
RPT = ACCR // NS
ROW_A = 1560
ROW_B = HALF - 15 * ROW_A

_mesh = plsc.VectorSubcoreMesh(
    core_axis_name="c", subcore_axis_name="s", num_cores=NC, num_subcores=NS
)
_sc_params = pltpu.CompilerParams(use_tc_tiling_on_sc=False)


def _zero_fill(buf, rows, width):
    per = width // 16

    def zf(i, _):
        buf[i // per, pl.ds((i % per) * 16, 16)] = jnp.zeros((16,), jnp.float32)
        return 0

    lax.fori_loop(0, rows * per, zf, 0)


def _zero_acc(acc, zbuf, s):
    def zc(i, _):
        pltpu.sync_copy(zbuf, acc.at[pl.ds(s * RPT + i * ZROWS, ZROWS)])
        return 0

    lax.fori_loop(0, RPT // ZROWS, zc, 0)


def _index_half(dstp, dstslab, idx2d, hbase, hoff, c_base):
    for sl in range(NSLAB):
        pltpu.sync_copy(dstp.at[pl.ds(hbase + sl * SLAB, SLAB)], dstslab)

        def jf(j, _, sl=sl):
            d = dstslab[pl.ds(j * 16, 16)]
            pos = hoff + sl * SLAB + j * 16 + lax.iota(jnp.int32, 16)
            loc = d - c_base
            ok = (loc >= 0) & (loc < HALF) & (pos < EPT)
            idx2d[sl * (SLAB // CH) + j // 8, pl.ds((j % 8) * 16, 16)] = (
                jnp.where(ok, loc, DUMMY)
            )
            return 0

        lax.fori_loop(0, SLAB // 16, jf, 0)


def _copy_out(acc, out, s, c_base, width):
    @pl.when(s < NS - 1)
    def _():
        pltpu.sync_copy(
            acc.at[pl.ds(s * ROW_A, ROW_A)],
            out.at[pl.ds(c_base + s * ROW_A, ROW_A)],
        )

    @pl.when(s == NS - 1)
    def _():
        pltpu.sync_copy(
            acc.at[pl.ds((NS - 1) * ROW_A, ROW_B)],
            out.at[pl.ds(c_base + (NS - 1) * ROW_A, ROW_B)],
        )


@functools.partial(
    pl.kernel,
    out_type=jax.ShapeDtypeStruct((N, 16), jnp.float32),
    mesh=_mesh,
    scratch_types=[
        pltpu.VMEM_SHARED((ACCR, 16), jnp.float32),
        pltpu.VMEM((ZROWS, 16), jnp.float32),
        pltpu.VMEM((CH, 16), jnp.float32),
        pltpu.VMEM((SLAB,), jnp.int32),
        pltpu.VMEM((HALFCH, CH), jnp.int32),
        pltpu.SemaphoreType.DMA,
        pltpu.SemaphoreType.DMA,
    ],
    compiler_params=_sc_params,
)
def _sc_deg(dstp, out, acc, zbuf, onesb, dstslab, idx2d, sm0, sm1):
    c = lax.axis_index("c")
    s = lax.axis_index("s")
    c_base = c * HALF
    _zero_fill(zbuf, ZROWS, 16)
    _zero_acc(acc, zbuf, s)

    def of(i, _):
        onesb[i, pl.ds(0, 16)] = jnp.full((16,), 1.0, jnp.float32)
        return 0

    lax.fori_loop(0, CH, of, 0)
    plsc.subcore_barrier()

    sems = [sm0, sm1]
    tile_base = s * EPT
    for h in range(2):
        _index_half(dstp, dstslab, idx2d, tile_base + h * HEDGES,
                    h * HEDGES, c_base)
        for b in range(2):
            pltpu.async_copy(onesb, acc.at[idx2d.at[b]], sems[b], add=True)

        def grp(k, _):
            for b in range(2):
                cc = k * 2 + b
                pltpu.make_async_copy(
                    onesb, acc.at[idx2d.at[cc]], sems[b]).wait()

                @pl.when(cc + 2 < HALFCH)
                def _():
                    pltpu.async_copy(
                        onesb, acc.at[idx2d.at[cc + 2]], sems[b], add=True)

            return 0

        lax.fori_loop(0, HALFCH // 2, grp, 0)
    plsc.subcore_barrier()
    _copy_out(acc, out, s, c_base, 16)


@functools.partial(
    pl.kernel,
    out_type=jax.ShapeDtypeStruct((N, H), jnp.float32),
    mesh=_mesh,
    scratch_types=[
        pltpu.VMEM_SHARED((ACCR, H), jnp.float32),
        pltpu.VMEM((ZROWS, H), jnp.float32),
        pltpu.VMEM((2 * SCH,), jnp.int32),
        pltpu.VMEM((2 * SCH,), jnp.int32),
        pltpu.VMEM((2 * SCH,), jnp.int32),
        pltpu.VMEM((2 * SCH,), jnp.int32),
        pltpu.VMEM((SCH,), jnp.int32),
        pltpu.VMEM((SCH,), jnp.int32),
        pltpu.VMEM((SCH,), jnp.int32),
        pltpu.VMEM((SCH,), jnp.int32),
        pltpu.VMEM((SCH, H), jnp.float32),
        pltpu.VMEM((SCH, H), jnp.float32),
        pltpu.VMEM((SCH, H), jnp.float32),
        pltpu.VMEM((SCH, H), jnp.float32),
        pltpu.SemaphoreType.DMA,
        pltpu.SemaphoreType.DMA,
        pltpu.SemaphoreType.DMA,
        pltpu.SemaphoreType.DMA,
        pltpu.SemaphoreType.DMA,
        pltpu.SemaphoreType.DMA,
        pltpu.SemaphoreType.DMA,
        pltpu.SemaphoreType.DMA,
        pltpu.SemaphoreType.DMA,
        pltpu.SemaphoreType.DMA,
        pltpu.SemaphoreType.DMA,
        pltpu.SemaphoreType.DMA,
    ],
    compiler_params=_sc_params,
)
def _sc_scatter(hh, epk, out, acc, zbuf, sd0, sd1, sd2, sd3,
                ix0, ix1, ix2, ix3, st0, st1, st2, st3,
                e0, e1, e2, e3, g0, g1, g2, g3, s0, s1, s2, s3):
    c = lax.axis_index("c")
    s = lax.axis_index("s")
    c_base = c * HALF
    _zero_fill(zbuf, ZROWS, H)
    _zero_acc(acc, zbuf, s)
    plsc.subcore_barrier()

    sdbuf = [sd0, sd1, sd2, sd3]
    idxb = [ix0, ix1, ix2, ix3]
    stags = [st0, st1, st2, st3]
    esems = [e0, e1, e2, e3]
    gsems = [g0, g1, g2, g3]
    ssems = [s0, s1, s2, s3]
    tile_base = s * SEPT
    ch_base = s * SNCH

    def eref(cc):
        return epk.at[pl.ds((ch_base + cc) * (2 * SCH), 2 * SCH)]

    def issue_e(cc, b):
        pltpu.async_copy(eref(cc), sdbuf[b], esems[b])

    def wait_e(cc, b):
        pltpu.make_async_copy(eref(cc), sdbuf[b], esems[b]).wait()

    def gref(b):
        return hh.at[sdbuf[b].at[pl.ds(0, SCH)]]

    def sref(b):
        return acc.at[idxb[b]]

    def compute_idx(cc, b):
        def jf(j, _):
            d = sdbuf[b][pl.ds(SCH + j * 16, 16)]
            pos = tile_base + cc * SCH + j * 16 + lax.iota(jnp.int32, 16)
            loc = d - c_base
            ok = (loc >= 0) & (loc < HALF) & (pos < E)
            idxb[b][pl.ds(j * 16, 16)] = jnp.where(ok, loc, DUMMY)
            return 0

        lax.fori_loop(0, SCH // 16, jf, 0)

    def advance(cc, b):
        wait_e(cc, b)
        pltpu.make_async_copy(stags[b], sref(b), ssems[b]).wait()
        compute_idx(cc, b)
        pltpu.async_copy(gref(b), stags[b], gsems[b])

    for b in range(4):
        issue_e(b, b)
    for cc in range(2):
        wait_e(cc, cc)
        compute_idx(cc, cc)
        pltpu.async_copy(gref(cc), stags[cc], gsems[cc])

    def grp(k, _):
        for b in range(4):
            cc = k * 4 + b
            b2 = (b + 2) % 4
            pltpu.make_async_copy(gref(b), stags[b], gsems[b]).wait()
            pltpu.async_copy(stags[b], sref(b), ssems[b], add=True)

            @pl.when(cc + 4 < SNCH)
            def _():
                issue_e(cc + 4, b)

            @pl.when(cc + 2 < SNCH)
            def _():
                @pl.when(cc >= 2)
                def _():
                    advance(cc + 2, b2)

                @pl.when(cc < 2)
                def _():
                    wait_e(cc + 2, b2)
                    compute_idx(cc + 2, b2)
                    pltpu.async_copy(gref(b2), stags[b2], gsems[b2])

        return 0

    lax.fori_loop(0, SNCH // 4, grp, 0)
    for cc in range(SNCH - 4, SNCH):
        b = cc % 4
        pltpu.make_async_copy(stags[b], sref(b), ssems[b]).wait()
    plsc.subcore_barrier()
    _copy_out(acc, out, s, c_base, H)


B = 1000
G = N // B


def _full(shape):
    return pl.BlockSpec(shape, lambda g: tuple(0 for _ in shape))


def _rows(shape):
    return pl.BlockSpec(shape, lambda g: (g,) + tuple(0 for _ in shape[1:]))


def _p0_body(x_ref, win_ref, bin_ref, w1_ref, deg_ref, h_ref, hh_ref):
    x0 = jnp.dot(x_ref[...], win_ref[...], preferred_element_type=jnp.float32)
    x0 = x0 + bin_ref[...]
    h = jnp.dot(x0, w1_ref[...], preferred_element_type=jnp.float32)
    dinv = lax.rsqrt(deg_ref[:, 0:1] + 1.0)
    h_ref[...] = h
    hh_ref[...] = h * dinv


def _p0(Xl, W_in, b_in, W1, deg):
    return pl.pallas_call(
        _p0_body,
        grid=(G,),
        in_specs=[
            _rows((B, IN_DIM)),
            _full((IN_DIM, H)),
            _full((1, H)),
            _full((H, H)),
            _rows((B, 16)),
        ],
        out_specs=[_rows((B, H)), _rows((B, H))],
        out_shape=[
            jax.ShapeDtypeStruct((N, H), jnp.float32),
            jax.ShapeDtypeStruct((N, H), jnp.float32),
        ],
    )(Xl, W_in, b_in, W1, deg)


def _p1_body(agg_ref, h_ref, deg_ref, b_ref, y_ref, sum_ref, ssq_ref, sa, sq):
    g = pl.program_id(0)
    dinv = lax.rsqrt(deg_ref[:, 0:1] + 1.0)
    y = dinv * agg_ref[...] + dinv * dinv * h_ref[...] + b_ref[...]
    y_ref[...] = y
    ps = jnp.sum(y.reshape(B // 8, 8, H), axis=0)
    pq = jnp.sum((y * y).reshape(B // 8, 8, H), axis=0)

    @pl.when(g == 0)
    def _():
        sa[...] = ps
        sq[...] = pq

    @pl.when(g > 0)
    def _():
        sa[...] += ps
        sq[...] += pq

    @pl.when(g == G - 1)
    def _():
        sum_ref[...] = jnp.sum(sa[...], axis=0, keepdims=True)
        ssq_ref[...] = jnp.sum(sq[...], axis=0, keepdims=True)


def _p1(agg, h, deg, b):
    return pl.pallas_call(
        _p1_body,
        grid=(G,),
        in_specs=[_rows((B, H)), _rows((B, H)), _rows((B, 16)), _full((1, H))],
        out_specs=[_rows((B, H)), _full((1, H)), _full((1, H))],
        out_shape=[
            jax.ShapeDtypeStruct((N, H), jnp.float32),
            jax.ShapeDtypeStruct((1, H), jnp.float32),
            jax.ShapeDtypeStruct((1, H), jnp.float32),
        ],
        scratch_shapes=[
            pltpu.VMEM((8, H), jnp.float32),
            pltpu.VMEM((8, H), jnp.float32),
        ],
    )(agg, h, deg, b)


def _bn(y, sum_v, ssq_v, gam, bet):
    m = sum_v * (1.0 / N)
    var = ssq_v * (1.0 / N) - m * m
    return (y - m) * lax.rsqrt(var + 1e-5) * gam + bet


def _p2_body(y_ref, sum_ref, ssq_ref, g_ref, be_ref, w2_ref, deg_ref,
             h_ref, hh_ref):
    xb = _bn(y_ref[...], sum_ref[...], ssq_ref[...], g_ref[...], be_ref[...])
    xb = jnp.maximum(xb, 0.0)
    h = jnp.dot(xb, w2_ref[...], preferred_element_type=jnp.float32)
    dinv = lax.rsqrt(deg_ref[:, 0:1] + 1.0)
    h_ref[...] = h
    hh_ref[...] = h * dinv


def _p2(y, sum_v, ssq_v, gam, bet, W2, deg):
    return pl.pallas_call(
        _p2_body,
        grid=(G,),
        in_specs=[
            _rows((B, H)),
            _full((1, H)),
            _full((1, H)),
            _full((1, H)),
            _full((1, H)),
            _full((H, H)),
            _rows((B, 16)),
        ],
        out_specs=[_rows((B, H)), _rows((B, H))],
        out_shape=[
            jax.ShapeDtypeStruct((N, H), jnp.float32),
            jax.ShapeDtypeStruct((N, H), jnp.float32),
        ],
    )(y, sum_v, ssq_v, gam, bet, W2, deg)


def _p4_body(y_ref, sum_ref, ssq_ref, g_ref, be_ref, wo1_ref, bo1_ref,
             wo2_ref, bo2_ref, hid_ref, log_ref):
    hid = _bn(y_ref[...], sum_ref[...], ssq_ref[...], g_ref[...], be_ref[...])
    hf = jnp.dot(hid, wo1_ref[...], preferred_element_type=jnp.float32)
    hf = jnp.maximum(hf + bo1_ref[...], 0.0)
    logit = jnp.sum(hf * wo2_ref[...], axis=1, keepdims=True) + bo2_ref[0, 0]
    hid_ref[...] = hid
    log_ref[...] = logit


def _p4(y, sum_v, ssq_v, gam, bet, Wo1, bo1, wo2t, bo2):
    return pl.pallas_call(
        _p4_body,
        grid=(G,),
        in_specs=[
            _rows((B, H)),
            _full((1, H)),
            _full((1, H)),
            _full((1, H)),
            _full((1, H)),
            _full((H, H)),
            _full((1, H)),
            _full((1, H)),
            _full((1, 1)),
        ],
        out_specs=[_rows((B, H)), _rows((B, 1))],
        out_shape=[
            jax.ShapeDtypeStruct((N, H), jnp.float32),
            jax.ShapeDtypeStruct((N, 1), jnp.float32),
        ],
    )(y, sum_v, ssq_v, gam, bet, Wo1, bo1, wo2t, bo2)


@jax.jit
def _run(X, edge_index, W_in, b_in, W1, b1, g1, beta1, W2, b2, g2, beta2,
         Wo1, bo1, Wo2, bo2):
    Xl = X[:, :, -1]
    dstp = jnp.concatenate(
        [edge_index[1], jnp.zeros((EPAD - E,), jnp.int32)])
    padk = jnp.zeros((SEPAD - E,), jnp.int32)
    srcpk = jnp.concatenate([edge_index[0], padk]).reshape(-1, SCH)
    dstpk = jnp.concatenate([edge_index[1], padk]).reshape(-1, SCH)
    epk = jnp.stack([srcpk, dstpk], axis=1).reshape(-1)

    deg = _sc_deg(dstp)

    h1, hh1 = _p0(Xl, W_in, b_in.reshape(1, H), W1, deg)
    agg1 = _sc_scatter(hh1, epk)
    y1, s1, q1 = _p1(agg1, h1, deg, b1.reshape(1, H))
    h2, hh2 = _p2(y1, s1, q1, g1.reshape(1, H), beta1.reshape(1, H), W2, deg)
    agg2 = _sc_scatter(hh2, epk)
    y2, s2, q2 = _p1(agg2, h2, deg, b2.reshape(1, H))
    hidden, logits = _p4(
        y2, s2, q2, g2.reshape(1, H), beta2.reshape(1, H),
        Wo1, bo1.reshape(1, H), Wo2.reshape(1, H), bo2.reshape(1, 1),
    )
    return logits, logits, hidden


def kernel(X, edge_index, W_in, b_in, W1, b1, g1, beta1, W2, b2, g2, beta2,
           Wo1, bo1, Wo2, bo2):
    return _run(X, edge_index, W_in, b_in, W1, b1, g1, beta1, W2, b2, g2,
                beta2, Wo1, bo1, Wo2, bo2)


# speedup vs baseline: 1.2067x; 1.2067x over previous
import functools

import jax
import jax.numpy as jnp
from jax import lax
from jax.experimental import pallas as pl
from jax.experimental.pallas import tpu as pltpu
from jax.experimental.pallas import tpu_sc as plsc

N = 50000
IN_DIM = 128
H = 64
E = 800000

NC = 2
NS = 16
CH = 128
EPT2 = E // NC // NS
DHALFCH = 196
DHED = DHALFCH * CH
SLAB = 3584
NSLAB = DHED // SLAB
EPAD = E + 2 * CH
ACC2 = 50048
RPT2 = ACC2 // NS
ROW_DA = 3120
ROW_DB = N - 15 * ROW_DA
SCH = 96
SNCH = 524
SEPT = SNCH * SCH
SEPAD = NS * SEPT
HALF = N // NC
ACCR = HALF + 24
DUMMY = HALF
ZROWS = 46
RPT = ACCR // NS
ROW_A = 1560
ROW_B = HALF - 15 * ROW_A

_mesh = plsc.VectorSubcoreMesh(
    core_axis_name="c", subcore_axis_name="s", num_cores=NC, num_subcores=NS
)
_sc_params = pltpu.CompilerParams(use_tc_tiling_on_sc=False)


def _zero_fill(buf, rows, width):
    per = width // 16

    def zf(i, _):
        buf[i // per, pl.ds((i % per) * 16, 16)] = jnp.zeros((16,), jnp.float32)
        return 0

    lax.fori_loop(0, rows * per, zf, 0)


def _zero_acc(acc, zbuf, s, rpt):
    def zc(i, _):
        pltpu.sync_copy(zbuf, acc.at[pl.ds(s * rpt + i * ZROWS, ZROWS)])
        return 0

    lax.fori_loop(0, rpt // ZROWS, zc, 0)


def _index_deg(dstp, dstslab, idx2d, hbase):
    for sl in range(NSLAB):
        pltpu.sync_copy(dstp.at[pl.ds(hbase + sl * SLAB, SLAB)], dstslab)

        def jf(j, _, sl=sl):
            d = dstslab[pl.ds(j * 16, 16)]
            pos = sl * SLAB + j * 16 + lax.iota(jnp.int32, 16)
            idx2d[sl * (SLAB // CH) + j // 8, pl.ds((j % 8) * 16, 16)] = (
                jnp.where(pos < EPT2, d, N)
            )
            return 0

        lax.fori_loop(0, SLAB // 16, jf, 0)


def _copy_out(acc, out, s, base, ra, rb):
    @pl.when(s < NS - 1)
    def _():
        pltpu.sync_copy(
            acc.at[pl.ds(s * ra, ra)],
            out.at[pl.ds(base + s * ra, ra)],
        )

    @pl.when(s == NS - 1)
    def _():
        pltpu.sync_copy(
            acc.at[pl.ds((NS - 1) * ra, rb)],
            out.at[pl.ds(base + (NS - 1) * ra, rb)],
        )


@functools.partial(
    pl.kernel,
    out_type=jax.ShapeDtypeStruct((NC * N, 16), jnp.float32),
    mesh=_mesh,
    scratch_types=[
        pltpu.VMEM_SHARED((ACC2, 16), jnp.float32),
        pltpu.VMEM((ZROWS, 16), jnp.float32),
        pltpu.VMEM((CH, 16), jnp.float32),
        pltpu.VMEM((SLAB,), jnp.int32),
        pltpu.VMEM((DHALFCH, CH), jnp.int32),
        pltpu.SemaphoreType.DMA,
        pltpu.SemaphoreType.DMA,
    ],
    compiler_params=_sc_params,
)
def _sc_deg(dstp, out, acc, zbuf, onesb, dstslab, idx2d, sm0, sm1):
    c = lax.axis_index("c")
    s = lax.axis_index("s")
    _zero_fill(zbuf, ZROWS, 16)
    _zero_acc(acc, zbuf, s, RPT2)

    def of(i, _):
        onesb[i, pl.ds(0, 16)] = jnp.full((16,), 1.0, jnp.float32)
        return 0

    lax.fori_loop(0, CH, of, 0)
    plsc.subcore_barrier()

    sems = [sm0, sm1]
    _index_deg(dstp, dstslab, idx2d, c * (E // NC) + s * EPT2)
    for b in range(2):
        pltpu.async_copy(onesb, acc.at[idx2d.at[b]], sems[b], add=True)

    def grp(k, _):
        for b in range(2):
            cc = k * 2 + b
            pltpu.make_async_copy(
                onesb, acc.at[idx2d.at[cc]], sems[b]).wait()

            @pl.when(cc + 2 < DHALFCH)
            def _():
                pltpu.async_copy(
                    onesb, acc.at[idx2d.at[cc + 2]], sems[b], add=True)

        return 0

    lax.fori_loop(0, DHALFCH // 2, grp, 0)
    plsc.subcore_barrier()
    _copy_out(acc, out, s, c * N, ROW_DA, ROW_DB)


@functools.partial(
    pl.kernel,
    out_type=jax.ShapeDtypeStruct((N, H), jnp.float32),
    mesh=_mesh,
    scratch_types=[
        pltpu.VMEM_SHARED((ACCR, H), jnp.float32),
        pltpu.VMEM((ZROWS, H), jnp.float32),
        pltpu.VMEM((2 * SCH,), jnp.int32),
        pltpu.VMEM((2 * SCH,), jnp.int32),
        pltpu.VMEM((2 * SCH,), jnp.int32),
        pltpu.VMEM((2 * SCH,), jnp.int32),
        pltpu.VMEM((SCH,), jnp.int32),
        pltpu.VMEM((SCH,), jnp.int32),
        pltpu.VMEM((SCH,), jnp.int32),
        pltpu.VMEM((SCH,), jnp.int32),
        pltpu.VMEM((SCH, H), jnp.float32),
        pltpu.VMEM((SCH, H), jnp.float32),
        pltpu.VMEM((SCH, H), jnp.float32),
        pltpu.VMEM((SCH, H), jnp.float32),
        pltpu.SemaphoreType.DMA,
        pltpu.SemaphoreType.DMA,
        pltpu.SemaphoreType.DMA,
        pltpu.SemaphoreType.DMA,
        pltpu.SemaphoreType.DMA,
        pltpu.SemaphoreType.DMA,
        pltpu.SemaphoreType.DMA,
        pltpu.SemaphoreType.DMA,
        pltpu.SemaphoreType.DMA,
        pltpu.SemaphoreType.DMA,
        pltpu.SemaphoreType.DMA,
        pltpu.SemaphoreType.DMA,
    ],
    compiler_params=_sc_params,
)
def _sc_scatter(hh, epk, out, acc, zbuf, sd0, sd1, sd2, sd3,
                ix0, ix1, ix2, ix3, st0, st1, st2, st3,
                e0, e1, e2, e3, g0, g1, g2, g3, s0, s1, s2, s3):
    c = lax.axis_index("c")
    s = lax.axis_index("s")
    c_base = c * HALF
    _zero_fill(zbuf, ZROWS, H)
    _zero_acc(acc, zbuf, s, RPT)
    plsc.subcore_barrier()

    sdbuf = [sd0, sd1, sd2, sd3]
    idxb = [ix0, ix1, ix2, ix3]
    stags = [st0, st1, st2, st3]
    esems = [e0, e1, e2, e3]
    gsems = [g0, g1, g2, g3]
    ssems = [s0, s1, s2, s3]
    tile_base = s * SEPT
    ch_base = s * SNCH

    def eref(cc):
        return epk.at[pl.ds((ch_base + cc) * (2 * SCH), 2 * SCH)]

    def issue_e(cc, b):
        pltpu.async_copy(eref(cc), sdbuf[b], esems[b])

    def wait_e(cc, b):
        pltpu.make_async_copy(eref(cc), sdbuf[b], esems[b]).wait()

    def gref(b):
        return hh.at[sdbuf[b].at[pl.ds(0, SCH)]]

    def sref(b):
        return acc.at[idxb[b]]

    def compute_idx(cc, b):
        def jf(j, _):
            d = sdbuf[b][pl.ds(SCH + j * 16, 16)]
            pos = tile_base + cc * SCH + j * 16 + lax.iota(jnp.int32, 16)
            loc = d - c_base
            ok = (loc >= 0) & (loc < HALF) & (pos < E)
            idxb[b][pl.ds(j * 16, 16)] = jnp.where(ok, loc, DUMMY)
            return 0

        lax.fori_loop(0, SCH // 16, jf, 0)

    def advance(cc, b):
        wait_e(cc, b)
        pltpu.make_async_copy(stags[b], sref(b), ssems[b]).wait()
        compute_idx(cc, b)
        pltpu.async_copy(gref(b), stags[b], gsems[b])

    for b in range(4):
        issue_e(b, b)
    for cc in range(2):
        wait_e(cc, cc)
        compute_idx(cc, cc)
        pltpu.async_copy(gref(cc), stags[cc], gsems[cc])

    def grp(k, _):
        for b in range(4):
            cc = k * 4 + b
            b2 = (b + 2) % 4
            pltpu.make_async_copy(gref(b), stags[b], gsems[b]).wait()
            pltpu.async_copy(stags[b], sref(b), ssems[b], add=True)

            @pl.when(cc + 4 < SNCH)
            def _():
                issue_e(cc + 4, b)

            @pl.when(cc + 2 < SNCH)
            def _():
                @pl.when(cc >= 2)
                def _():
                    advance(cc + 2, b2)

                @pl.when(cc < 2)
                def _():
                    wait_e(cc + 2, b2)
                    compute_idx(cc + 2, b2)
                    pltpu.async_copy(gref(b2), stags[b2], gsems[b2])

        return 0

    lax.fori_loop(0, SNCH // 4, grp, 0)
    for cc in range(SNCH - 4, SNCH):
        b = cc % 4
        pltpu.make_async_copy(stags[b], sref(b), ssems[b]).wait()
    plsc.subcore_barrier()
    _copy_out(acc, out, s, c_base, ROW_A, ROW_B)


B = 1000
G = N // B


def _full(shape):
    return pl.BlockSpec(shape, lambda g: tuple(0 for _ in shape))


def _rows(shape):
    return pl.BlockSpec(shape, lambda g: (g,) + tuple(0 for _ in shape[1:]))


def _degb_spec():
    return pl.BlockSpec((B, 16), lambda g: (g + G, 0))


def _dinv(dega_ref, degb_ref):
    return lax.rsqrt(dega_ref[:, 0:1] + degb_ref[:, 0:1] + 1.0)


def _p0_body(x_ref, win_ref, bin_ref, w1_ref, dega_ref, degb_ref,
             h_ref, hh_ref):
    x0 = jnp.dot(x_ref[...], win_ref[...], preferred_element_type=jnp.float32)
    x0 = x0 + bin_ref[...]
    h = jnp.dot(x0, w1_ref[...], preferred_element_type=jnp.float32)
    dinv = _dinv(dega_ref, degb_ref)
    h_ref[...] = h
    hh_ref[...] = h * dinv


def _p0(Xl, W_in, b_in, W1, deg):
    return pl.pallas_call(
        _p0_body,
        grid=(G,),
        in_specs=[
            _rows((B, IN_DIM)),
            _full((IN_DIM, H)),
            _full((1, H)),
            _full((H, H)),
            _rows((B, 16)),
            _degb_spec(),
        ],
        out_specs=[_rows((B, H)), _rows((B, H))],
        out_shape=[
            jax.ShapeDtypeStruct((N, H), jnp.float32),
            jax.ShapeDtypeStruct((N, H), jnp.float32),
        ],
    )(Xl, W_in, b_in, W1, deg, deg)


def _p1_body(agg_ref, h_ref, dega_ref, degb_ref, b_ref,
             y_ref, sum_ref, ssq_ref, sa, sq):
    g = pl.program_id(0)
    dinv = _dinv(dega_ref, degb_ref)
    y = dinv * agg_ref[...] + dinv * dinv * h_ref[...] + b_ref[...]
    y_ref[...] = y
    ps = jnp.sum(y.reshape(B // 8, 8, H), axis=0)
    pq = jnp.sum((y * y).reshape(B // 8, 8, H), axis=0)

    @pl.when(g == 0)
    def _():
        sa[...] = ps
        sq[...] = pq

    @pl.when(g > 0)
    def _():
        sa[...] += ps
        sq[...] += pq

    @pl.when(g == G - 1)
    def _():
        sum_ref[...] = jnp.sum(sa[...], axis=0, keepdims=True)
        ssq_ref[...] = jnp.sum(sq[...], axis=0, keepdims=True)


def _p1(agg, h, deg, b):
    return pl.pallas_call(
        _p1_body,
        grid=(G,),
        in_specs=[_rows((B, H)), _rows((B, H)), _rows((B, 16)),
                  _degb_spec(), _full((1, H))],
        out_specs=[_rows((B, H)), _full((1, H)), _full((1, H))],
        out_shape=[
            jax.ShapeDtypeStruct((N, H), jnp.float32),
            jax.ShapeDtypeStruct((1, H), jnp.float32),
            jax.ShapeDtypeStruct((1, H), jnp.float32),
        ],
        scratch_shapes=[
            pltpu.VMEM((8, H), jnp.float32),
            pltpu.VMEM((8, H), jnp.float32),
        ],
    )(agg, h, deg, deg, b)


def _bn(y, sum_v, ssq_v, gam, bet):
    m = sum_v * (1.0 / N)
    var = ssq_v * (1.0 / N) - m * m
    return (y - m) * lax.rsqrt(var + 1e-5) * gam + bet


def _p2_body(y_ref, sum_ref, ssq_ref, g_ref, be_ref, w2_ref, dega_ref,
             degb_ref, h_ref, hh_ref):
    xb = _bn(y_ref[...], sum_ref[...], ssq_ref[...], g_ref[...], be_ref[...])
    xb = jnp.maximum(xb, 0.0)
    h = jnp.dot(xb, w2_ref[...], preferred_element_type=jnp.float32)
    dinv = _dinv(dega_ref, degb_ref)
    h_ref[...] = h
    hh_ref[...] = h * dinv


def _p2(y, sum_v, ssq_v, gam, bet, W2, deg):
    return pl.pallas_call(
        _p2_body,
        grid=(G,),
        in_specs=[
            _rows((B, H)),
            _full((1, H)),
            _full((1, H)),
            _full((1, H)),
            _full((1, H)),
            _full((H, H)),
            _rows((B, 16)),
            _degb_spec(),
        ],
        out_specs=[_rows((B, H)), _rows((B, H))],
        out_shape=[
            jax.ShapeDtypeStruct((N, H), jnp.float32),
            jax.ShapeDtypeStruct((N, H), jnp.float32),
        ],
    )(y, sum_v, ssq_v, gam, bet, W2, deg, deg)


def _p4_body(y_ref, sum_ref, ssq_ref, g_ref, be_ref, wo1_ref, bo1_ref,
             wo2_ref, bo2_ref, hid_ref, log_ref):
    hid = _bn(y_ref[...], sum_ref[...], ssq_ref[...], g_ref[...], be_ref[...])
    hf = jnp.dot(hid, wo1_ref[...], preferred_element_type=jnp.float32)
    hf = jnp.maximum(hf + bo1_ref[...], 0.0)
    logit = jnp.sum(hf * wo2_ref[...], axis=1, keepdims=True) + bo2_ref[0, 0]
    hid_ref[...] = hid
    log_ref[...] = logit


def _p4(y, sum_v, ssq_v, gam, bet, Wo1, bo1, wo2t, bo2):
    return pl.pallas_call(
        _p4_body,
        grid=(G,),
        in_specs=[
            _rows((B, H)),
            _full((1, H)),
            _full((1, H)),
            _full((1, H)),
            _full((1, H)),
            _full((H, H)),
            _full((1, H)),
            _full((1, H)),
            _full((1, 1)),
        ],
        out_specs=[_rows((B, H)), _rows((B, 1))],
        out_shape=[
            jax.ShapeDtypeStruct((N, H), jnp.float32),
            jax.ShapeDtypeStruct((N, 1), jnp.float32),
        ],
    )(y, sum_v, ssq_v, gam, bet, Wo1, bo1, wo2t, bo2)


@jax.jit
def _run(X, edge_index, W_in, b_in, W1, b1, g1, beta1, W2, b2, g2, beta2,
         Wo1, bo1, Wo2, bo2):
    Xl = X[:, :, -1]
    dstp = jnp.concatenate(
        [edge_index[1], jnp.zeros((EPAD - E,), jnp.int32)])
    padk = jnp.zeros((SEPAD - E,), jnp.int32)
    srcpk = jnp.concatenate([edge_index[0], padk]).reshape(-1, SCH)
    dstpk = jnp.concatenate([edge_index[1], padk]).reshape(-1, SCH)
    epk = jnp.stack([srcpk, dstpk], axis=1).reshape(-1)

    deg = _sc_deg(dstp)

    h1, hh1 = _p0(Xl, W_in, b_in.reshape(1, H), W1, deg)
    agg1 = _sc_scatter(hh1, epk)
    y1, s1, q1 = _p1(agg1, h1, deg, b1.reshape(1, H))
    h2, hh2 = _p2(y1, s1, q1, g1.reshape(1, H), beta1.reshape(1, H), W2, deg)
    agg2 = _sc_scatter(hh2, epk)
    y2, s2, q2 = _p1(agg2, h2, deg, b2.reshape(1, H))
    hidden, logits = _p4(
        y2, s2, q2, g2.reshape(1, H), beta2.reshape(1, H),
        Wo1, bo1.reshape(1, H), Wo2.reshape(1, H), bo2.reshape(1, 1),
    )
    return logits, logits, hidden


def kernel(X, edge_index, W_in, b_in, W1, b1, g1, beta1, W2, b2, g2, beta2,
           Wo1, bo1, Wo2, bo2):
    return _run(X, edge_index, W_in, b_in, W1, b1, g1, beta1, W2, b2, g2,
                beta2, Wo1, bo1, Wo2, bo2)


# speedup vs baseline: 1.9077x; 1.5810x over previous
import functools

import jax
import jax.numpy as jnp
from jax import lax
from jax.experimental import pallas as pl
from jax.experimental.pallas import tpu as pltpu
from jax.experimental.pallas import tpu_sc as plsc

N = 50000
IN_DIM = 128
H = 64
E = 800000

NC = 2
NS = 16
CH = 128
EPT2 = E // NC // NS
DHALFCH = 196
DHED = DHALFCH * CH
SLAB = 3584
NSLAB = DHED // SLAB
EPAD = E + 2 * CH
ACC2 = 50048
RPT2 = ACC2 // NS
ROW_DA = 3120
ROW_DB = N - 15 * ROW_DA
SCH = 96
SNCH = 524
SEPT = SNCH * SCH
SEPAD = NS * SEPT
HW = H // NC
ZROWS = 46

_mesh = plsc.VectorSubcoreMesh(
    core_axis_name="c", subcore_axis_name="s", num_cores=NC, num_subcores=NS
)
_sc_params = pltpu.CompilerParams(use_tc_tiling_on_sc=False)


def _zero_fill(buf, rows, width):
    per = width // 16

    def zf(i, _):
        buf[i // per, pl.ds((i % per) * 16, 16)] = jnp.zeros((16,), jnp.float32)
        return 0

    lax.fori_loop(0, rows * per, zf, 0)


def _zero_acc(acc, zbuf, s, rpt):
    def zc(i, _):
        pltpu.sync_copy(zbuf, acc.at[pl.ds(s * rpt + i * ZROWS, ZROWS)])
        return 0

    lax.fori_loop(0, rpt // ZROWS, zc, 0)


def _index_deg(dstp, dstslab, idx2d, hbase):
    for sl in range(NSLAB):
        pltpu.sync_copy(dstp.at[pl.ds(hbase + sl * SLAB, SLAB)], dstslab)

        def jf(j, _, sl=sl):
            d = dstslab[pl.ds(j * 16, 16)]
            pos = sl * SLAB + j * 16 + lax.iota(jnp.int32, 16)
            idx2d[sl * (SLAB // CH) + j // 8, pl.ds((j % 8) * 16, 16)] = (
                jnp.where(pos < EPT2, d, N)
            )
            return 0

        lax.fori_loop(0, SLAB // 16, jf, 0)


def _copy_out(acc, out, s, base, ra, rb):
    @pl.when(s < NS - 1)
    def _():
        pltpu.sync_copy(
            acc.at[pl.ds(s * ra, ra)],
            out.at[pl.ds(base + s * ra, ra)],
        )

    @pl.when(s == NS - 1)
    def _():
        pltpu.sync_copy(
            acc.at[pl.ds((NS - 1) * ra, rb)],
            out.at[pl.ds(base + (NS - 1) * ra, rb)],
        )


@functools.partial(
    pl.kernel,
    out_type=jax.ShapeDtypeStruct((NC * N, 16), jnp.float32),
    mesh=_mesh,
    scratch_types=[
        pltpu.VMEM_SHARED((ACC2, 16), jnp.float32),
        pltpu.VMEM((ZROWS, 16), jnp.float32),
        pltpu.VMEM((CH, 16), jnp.float32),
        pltpu.VMEM((SLAB,), jnp.int32),
        pltpu.VMEM((DHALFCH, CH), jnp.int32),
        pltpu.SemaphoreType.DMA,
        pltpu.SemaphoreType.DMA,
    ],
    compiler_params=_sc_params,
)
def _sc_deg(dstp, out, acc, zbuf, onesb, dstslab, idx2d, sm0, sm1):
    c = lax.axis_index("c")
    s = lax.axis_index("s")
    _zero_fill(zbuf, ZROWS, 16)
    _zero_acc(acc, zbuf, s, RPT2)

    def of(i, _):
        onesb[i, pl.ds(0, 16)] = jnp.full((16,), 1.0, jnp.float32)
        return 0

    lax.fori_loop(0, CH, of, 0)
    plsc.subcore_barrier()

    sems = [sm0, sm1]
    _index_deg(dstp, dstslab, idx2d, c * (E // NC) + s * EPT2)
    for b in range(2):
        pltpu.async_copy(onesb, acc.at[idx2d.at[b]], sems[b], add=True)

    def grp(k, _):
        for b in range(2):
            cc = k * 2 + b
            pltpu.make_async_copy(
                onesb, acc.at[idx2d.at[cc]], sems[b]).wait()

            @pl.when(cc + 2 < DHALFCH)
            def _():
                pltpu.async_copy(
                    onesb, acc.at[idx2d.at[cc + 2]], sems[b], add=True)

        return 0

    lax.fori_loop(0, DHALFCH // 2, grp, 0)
    plsc.subcore_barrier()
    _copy_out(acc, out, s, c * N, ROW_DA, ROW_DB)


@functools.partial(
    pl.kernel,
    out_type=jax.ShapeDtypeStruct((NC * N, HW), jnp.float32),
    mesh=_mesh,
    scratch_types=[
        pltpu.VMEM_SHARED((ACC2, HW), jnp.float32),
        pltpu.VMEM((ZROWS, HW), jnp.float32),
        pltpu.VMEM((2 * SCH,), jnp.int32),
        pltpu.VMEM((2 * SCH,), jnp.int32),
        pltpu.VMEM((2 * SCH,), jnp.int32),
        pltpu.VMEM((2 * SCH,), jnp.int32),
        pltpu.VMEM((SCH,), jnp.int32),
        pltpu.VMEM((SCH,), jnp.int32),
        pltpu.VMEM((SCH,), jnp.int32),
        pltpu.VMEM((SCH,), jnp.int32),
        pltpu.VMEM((SCH, HW), jnp.float32),
        pltpu.VMEM((SCH, HW), jnp.float32),
        pltpu.VMEM((SCH, HW), jnp.float32),
        pltpu.VMEM((SCH, HW), jnp.float32),
        pltpu.SemaphoreType.DMA,
        pltpu.SemaphoreType.DMA,
        pltpu.SemaphoreType.DMA,
        pltpu.SemaphoreType.DMA,
        pltpu.SemaphoreType.DMA,
        pltpu.SemaphoreType.DMA,
        pltpu.SemaphoreType.DMA,
        pltpu.SemaphoreType.DMA,
        pltpu.SemaphoreType.DMA,
        pltpu.SemaphoreType.DMA,
        pltpu.SemaphoreType.DMA,
        pltpu.SemaphoreType.DMA,
    ],
    compiler_params=_sc_params,
)
def _sc_scatter(hha, hhb, epk, out, acc, zbuf, sd0, sd1, sd2, sd3,
                ix0, ix1, ix2, ix3, st0, st1, st2, st3,
                e0, e1, e2, e3, g0, g1, g2, g3, s0, s1, s2, s3):
    c = lax.axis_index("c")
    s = lax.axis_index("s")
    _zero_fill(zbuf, ZROWS, HW)
    _zero_acc(acc, zbuf, s, RPT2)
    plsc.subcore_barrier()

    sdbuf = [sd0, sd1, sd2, sd3]
    idxb = [ix0, ix1, ix2, ix3]
    stags = [st0, st1, st2, st3]
    esems = [e0, e1, e2, e3]
    gsems = [g0, g1, g2, g3]
    ssems = [s0, s1, s2, s3]
    tile_base = s * SEPT
    ch_base = s * SNCH

    def eref(cc):
        return epk.at[pl.ds((ch_base + cc) * (2 * SCH), 2 * SCH)]

    def issue_e(cc, b):
        pltpu.async_copy(eref(cc), sdbuf[b], esems[b])

    def wait_e(cc, b):
        pltpu.make_async_copy(eref(cc), sdbuf[b], esems[b]).wait()

    def sref(b):
        return acc.at[idxb[b]]

    def compute_idx(cc, b):
        def jf(j, _):
            d = sdbuf[b][pl.ds(SCH + j * 16, 16)]
            pos = tile_base + cc * SCH + j * 16 + lax.iota(jnp.int32, 16)
            idxb[b][pl.ds(j * 16, 16)] = jnp.where(pos < E, d, N)
            return 0

        lax.fori_loop(0, SCH // 16, jf, 0)

    def pipe(hh):
        def gref(b):
            return hh.at[sdbuf[b].at[pl.ds(0, SCH)]]

        def advance(cc, b):
            wait_e(cc, b)
            pltpu.make_async_copy(stags[b], sref(b), ssems[b]).wait()
            compute_idx(cc, b)
            pltpu.async_copy(gref(b), stags[b], gsems[b])

        for b in range(4):
            issue_e(b, b)
        for cc in range(2):
            wait_e(cc, cc)
            compute_idx(cc, cc)
            pltpu.async_copy(gref(cc), stags[cc], gsems[cc])

        def grp(k, _):
            for b in range(4):
                cc = k * 4 + b
                b2 = (b + 2) % 4
                pltpu.make_async_copy(gref(b), stags[b], gsems[b]).wait()
                pltpu.async_copy(stags[b], sref(b), ssems[b], add=True)

                @pl.when(cc + 4 < SNCH)
                def _():
                    issue_e(cc + 4, b)

                @pl.when(cc + 2 < SNCH)
                def _():
                    @pl.when(cc >= 2)
                    def _():
                        advance(cc + 2, b2)

                    @pl.when(cc < 2)
                    def _():
                        wait_e(cc + 2, b2)
                        compute_idx(cc + 2, b2)
                        pltpu.async_copy(gref(b2), stags[b2], gsems[b2])

            return 0

        lax.fori_loop(0, SNCH // 4, grp, 0)
        for cc in range(SNCH - 4, SNCH):
            b = cc % 4
            pltpu.make_async_copy(stags[b], sref(b), ssems[b]).wait()

    @pl.when(c == 0)
    def _():
        pipe(hha)

    @pl.when(c == 1)
    def _():
        pipe(hhb)

    plsc.subcore_barrier()
    _copy_out(acc, out, s, c * N, ROW_DA, ROW_DB)


B = 1000
G = N // B


def _full(shape):
    return pl.BlockSpec(shape, lambda g: tuple(0 for _ in shape))


def _rows(shape):
    return pl.BlockSpec(shape, lambda g: (g,) + tuple(0 for _ in shape[1:]))


def _degb_spec():
    return pl.BlockSpec((B, 16), lambda g: (g + G, 0))


def _dinv(dega_ref, degb_ref):
    return lax.rsqrt(dega_ref[:, 0:1] + degb_ref[:, 0:1] + 1.0)


def _p0_body(x_ref, win_ref, bin_ref, w1_ref, dega_ref, degb_ref,
             h_ref, hha_ref, hhb_ref):
    x0 = jnp.dot(x_ref[...], win_ref[...], preferred_element_type=jnp.float32)
    x0 = x0 + bin_ref[...]
    h = jnp.dot(x0, w1_ref[...], preferred_element_type=jnp.float32)
    dinv = _dinv(dega_ref, degb_ref)
    h_ref[...] = h
    hh = h * dinv
    hha_ref[...] = hh[:, :HW]
    hhb_ref[...] = hh[:, HW:]


def _p0(Xl, W_in, b_in, W1, deg):
    return pl.pallas_call(
        _p0_body,
        grid=(G,),
        in_specs=[
            _rows((B, IN_DIM)),
            _full((IN_DIM, H)),
            _full((1, H)),
            _full((H, H)),
            _rows((B, 16)),
            _degb_spec(),
        ],
        out_specs=[_rows((B, H)), _rows((B, HW)), _rows((B, HW))],
        out_shape=[
            jax.ShapeDtypeStruct((N, H), jnp.float32),
            jax.ShapeDtypeStruct((N, HW), jnp.float32),
            jax.ShapeDtypeStruct((N, HW), jnp.float32),
        ],
    )(Xl, W_in, b_in, W1, deg, deg)


def _p1_body(agga_ref, aggb_ref, h_ref, dega_ref, degb_ref, b_ref,
             y_ref, sum_ref, ssq_ref, sa, sq):
    g = pl.program_id(0)
    dinv = _dinv(dega_ref, degb_ref)
    agg = jnp.concatenate([agga_ref[...], aggb_ref[...]], axis=1)
    y = dinv * agg + dinv * dinv * h_ref[...] + b_ref[...]
    y_ref[...] = y
    ps = jnp.sum(y.reshape(B // 8, 8, H), axis=0)
    pq = jnp.sum((y * y).reshape(B // 8, 8, H), axis=0)

    @pl.when(g == 0)
    def _():
        sa[...] = ps
        sq[...] = pq

    @pl.when(g > 0)
    def _():
        sa[...] += ps
        sq[...] += pq

    @pl.when(g == G - 1)
    def _():
        sum_ref[...] = jnp.sum(sa[...], axis=0, keepdims=True)
        ssq_ref[...] = jnp.sum(sq[...], axis=0, keepdims=True)


def _p1(agg, h, deg, b):
    return pl.pallas_call(
        _p1_body,
        grid=(G,),
        in_specs=[pl.BlockSpec((B, HW), lambda g: (g, 0)),
                  pl.BlockSpec((B, HW), lambda g: (g + G, 0)),
                  _rows((B, H)), _rows((B, 16)),
                  _degb_spec(), _full((1, H))],
        out_specs=[_rows((B, H)), _full((1, H)), _full((1, H))],
        out_shape=[
            jax.ShapeDtypeStruct((N, H), jnp.float32),
            jax.ShapeDtypeStruct((1, H), jnp.float32),
            jax.ShapeDtypeStruct((1, H), jnp.float32),
        ],
        scratch_shapes=[
            pltpu.VMEM((8, H), jnp.float32),
            pltpu.VMEM((8, H), jnp.float32),
        ],
    )(agg, agg, h, deg, deg, b)


def _bn(y, sum_v, ssq_v, gam, bet):
    m = sum_v * (1.0 / N)
    var = ssq_v * (1.0 / N) - m * m
    return (y - m) * lax.rsqrt(var + 1e-5) * gam + bet


def _p2_body(y_ref, sum_ref, ssq_ref, g_ref, be_ref, w2_ref, dega_ref,
             degb_ref, h_ref, hha_ref, hhb_ref):
    xb = _bn(y_ref[...], sum_ref[...], ssq_ref[...], g_ref[...], be_ref[...])
    xb = jnp.maximum(xb, 0.0)
    h = jnp.dot(xb, w2_ref[...], preferred_element_type=jnp.float32)
    dinv = _dinv(dega_ref, degb_ref)
    h_ref[...] = h
    hh = h * dinv
    hha_ref[...] = hh[:, :HW]
    hhb_ref[...] = hh[:, HW:]


def _p2(y, sum_v, ssq_v, gam, bet, W2, deg):
    return pl.pallas_call(
        _p2_body,
        grid=(G,),
        in_specs=[
            _rows((B, H)),
            _full((1, H)),
            _full((1, H)),
            _full((1, H)),
            _full((1, H)),
            _full((H, H)),
            _rows((B, 16)),
            _degb_spec(),
        ],
        out_specs=[_rows((B, H)), _rows((B, HW)), _rows((B, HW))],
        out_shape=[
            jax.ShapeDtypeStruct((N, H), jnp.float32),
            jax.ShapeDtypeStruct((N, HW), jnp.float32),
            jax.ShapeDtypeStruct((N, HW), jnp.float32),
        ],
    )(y, sum_v, ssq_v, gam, bet, W2, deg, deg)


def _p4_body(y_ref, sum_ref, ssq_ref, g_ref, be_ref, wo1_ref, bo1_ref,
             wo2_ref, bo2_ref, hid_ref, log_ref):
    hid = _bn(y_ref[...], sum_ref[...], ssq_ref[...], g_ref[...], be_ref[...])
    hf = jnp.dot(hid, wo1_ref[...], preferred_element_type=jnp.float32)
    hf = jnp.maximum(hf + bo1_ref[...], 0.0)
    logit = jnp.sum(hf * wo2_ref[...], axis=1, keepdims=True) + bo2_ref[0, 0]
    hid_ref[...] = hid
    log_ref[...] = logit


def _p4(y, sum_v, ssq_v, gam, bet, Wo1, bo1, wo2t, bo2):
    return pl.pallas_call(
        _p4_body,
        grid=(G,),
        in_specs=[
            _rows((B, H)),
            _full((1, H)),
            _full((1, H)),
            _full((1, H)),
            _full((1, H)),
            _full((H, H)),
            _full((1, H)),
            _full((1, H)),
            _full((1, 1)),
        ],
        out_specs=[_rows((B, H)), _rows((B, 1))],
        out_shape=[
            jax.ShapeDtypeStruct((N, H), jnp.float32),
            jax.ShapeDtypeStruct((N, 1), jnp.float32),
        ],
    )(y, sum_v, ssq_v, gam, bet, Wo1, bo1, wo2t, bo2)


@jax.jit
def _run(X, edge_index, W_in, b_in, W1, b1, g1, beta1, W2, b2, g2, beta2,
         Wo1, bo1, Wo2, bo2):
    Xl = X[:, :, -1]
    dstp = jnp.concatenate(
        [edge_index[1], jnp.zeros((EPAD - E,), jnp.int32)])
    padk = jnp.zeros((SEPAD - E,), jnp.int32)
    srcpk = jnp.concatenate([edge_index[0], padk]).reshape(-1, SCH)
    dstpk = jnp.concatenate([edge_index[1], padk]).reshape(-1, SCH)
    epk = jnp.stack([srcpk, dstpk], axis=1).reshape(-1)

    deg = _sc_deg(dstp)

    h1, hha1, hhb1 = _p0(Xl, W_in, b_in.reshape(1, H), W1, deg)
    agg1 = _sc_scatter(hha1, hhb1, epk)
    y1, s1, q1 = _p1(agg1, h1, deg, b1.reshape(1, H))
    h2, hha2, hhb2 = _p2(y1, s1, q1, g1.reshape(1, H), beta1.reshape(1, H),
                         W2, deg)
    agg2 = _sc_scatter(hha2, hhb2, epk)
    y2, s2, q2 = _p1(agg2, h2, deg, b2.reshape(1, H))
    hidden, logits = _p4(
        y2, s2, q2, g2.reshape(1, H), beta2.reshape(1, H),
        Wo1, bo1.reshape(1, H), Wo2.reshape(1, H), bo2.reshape(1, 1),
    )
    return logits, logits, hidden


def kernel(X, edge_index, W_in, b_in, W1, b1, g1, beta1, W2, b2, g2, beta2,
           Wo1, bo1, Wo2, bo2):
    return _run(X, edge_index, W_in, b_in, W1, b1, g1, beta1, W2, b2, g2,
                beta2, Wo1, bo1, Wo2, bo2)
